# baseline (speedup 1.0000x reference)
import functools

import jax
import jax.numpy as jnp
from jax import lax
from jax.experimental import pallas as pl
from jax.experimental.pallas import tpu as pltpu
from jax.experimental.pallas import tpu_sc as plsc

N = 10000
D = 128
G = 64
CH = 128
NCORE = 2
NSUB = 16
NW = NCORE * NSUB
ACC_PER_TILE = 640
ACC_ROWS = NSUB * ACC_PER_TILE
DEG_W = 128
RB = 2000
NBLK = N // RB

_sc_mesh = plsc.VectorSubcoreMesh(core_axis_name="c", subcore_axis_name="s")


def _cdiv(a, b):
    return (a + b - 1) // b


@functools.lru_cache(maxsize=None)
def _make_deg_kernel(cpt):
    @functools.partial(
        pl.kernel,
        out_type=jax.ShapeDtypeStruct((NCORE, ACC_ROWS, DEG_W), jnp.float32),
        mesh=_sc_mesh,
        scratch_types=[
            pltpu.VMEM((cpt, CH), jnp.int32),
            pltpu.VMEM((CH, DEG_W), jnp.float32),
            pltpu.VMEM_SHARED((ACC_ROWS, DEG_W), jnp.float32),
        ],
    )
    def deg_kernel(dst_hbm, ones_hbm, zeros_hbm, out_hbm, dst_v, ones_v, acc_sh):
        cid = lax.axis_index("c")
        sid = lax.axis_index("s")
        wid = cid * NSUB + sid
        row0 = sid * ACC_PER_TILE
        pltpu.sync_copy(zeros_hbm, acc_sh.at[pl.ds(row0, ACC_PER_TILE)])
        pltpu.sync_copy(ones_hbm, ones_v)
        pltpu.sync_copy(dst_hbm.at[wid], dst_v)
        plsc.subcore_barrier()

        def body(c, carry):
            pltpu.sync_copy(ones_v, acc_sh.at[dst_v.at[c]], add=True)
            return carry

        lax.fori_loop(0, cpt, body, 0)
        plsc.subcore_barrier()
        pltpu.sync_copy(acc_sh.at[pl.ds(row0, ACC_PER_TILE)],
                        out_hbm.at[cid, pl.ds(row0, ACC_PER_TILE)])

    return deg_kernel


@functools.lru_cache(maxsize=None)
def _make_scatter_kernel(cpt):
    @functools.partial(
        pl.kernel,
        out_type=jax.ShapeDtypeStruct((NCORE, ACC_ROWS, D), jnp.float32),
        mesh=_sc_mesh,
        scratch_types=[
            pltpu.VMEM((cpt, CH), jnp.int32),
            pltpu.VMEM((cpt, CH), jnp.int32),
            pltpu.VMEM((CH, D), jnp.float32),
            pltpu.VMEM_SHARED((ACC_ROWS, D), jnp.float32),
            pltpu.SemaphoreType.DMA,
        ],
    )
    def scatter_kernel(h_hbm, src_hbm, dst_hbm, zeros_hbm, out_hbm,
                       src_v, dst_v, rows_v, acc_sh, sem):
        cid = lax.axis_index("c")
        sid = lax.axis_index("s")
        wid = cid * NSUB + sid
        row0 = sid * ACC_PER_TILE
        pltpu.sync_copy(zeros_hbm, acc_sh.at[pl.ds(row0, ACC_PER_TILE)])
        pltpu.sync_copy(src_hbm.at[wid], src_v)
        pltpu.sync_copy(dst_hbm.at[wid], dst_v)
        plsc.subcore_barrier()

        def body(c, carry):
            pltpu.async_copy(h_hbm.at[src_v.at[c]], rows_v, sem).wait()
            pltpu.sync_copy(rows_v, acc_sh.at[dst_v.at[c]], add=True)
            return carry

        lax.fori_loop(0, cpt, body, 0)
        plsc.subcore_barrier()
        pltpu.sync_copy(acc_sh.at[pl.ds(row0, ACC_PER_TILE)],
                        out_hbm.at[cid, pl.ds(row0, ACC_PER_TILE)])

    return scatter_kernel


def _dinv_block(deg_ref):
    d = deg_ref[0] + deg_ref[1]
    return lax.rsqrt(d[:, 0:1] + 1.0)


def _tc_lin1(x, W1, deg):
    def body(x_ref, w_ref, deg_ref, out_ref):
        dinv = _dinv_block(deg_ref)
        h = jnp.dot(x_ref[...], w_ref[...], preferred_element_type=jnp.float32)
        out_ref[...] = h * dinv

    return pl.pallas_call(
        body,
        grid=(NBLK,),
        in_specs=[
            pl.BlockSpec((RB, D), lambda i: (i, 0)),
            pl.BlockSpec((D, D), lambda i: (0, 0)),
            pl.BlockSpec((NCORE, RB, DEG_W), lambda i: (0, i, 0)),
        ],
        out_specs=pl.BlockSpec((RB, D), lambda i: (i, 0)),
        out_shape=jax.ShapeDtypeStruct((N, D), jnp.float32),
    )(x, W1, deg)


def _tc_layer(acc, hprev, deg, b, W):
    def body(acc_ref, h_ref, deg_ref, b_ref, w_ref, out_ref):
        dinv = _dinv_block(deg_ref)
        agg = dinv * (acc_ref[0] + acc_ref[1] + h_ref[...]) + b_ref[...]
        h = jnp.maximum(agg, 0.0)
        out_ref[...] = dinv * jnp.dot(h, w_ref[...],
                                      preferred_element_type=jnp.float32)

    return pl.pallas_call(
        body,
        grid=(NBLK,),
        in_specs=[
            pl.BlockSpec((NCORE, RB, D), lambda i: (0, i, 0)),
            pl.BlockSpec((RB, D), lambda i: (i, 0)),
            pl.BlockSpec((NCORE, RB, DEG_W), lambda i: (0, i, 0)),
            pl.BlockSpec((1, D), lambda i: (0, 0)),
            pl.BlockSpec((D, D), lambda i: (0, 0)),
        ],
        out_specs=pl.BlockSpec((RB, D), lambda i: (i, 0)),
        out_shape=jax.ShapeDtypeStruct((N, D), jnp.float32),
    )(acc, hprev, deg, b, W)


def _tc_final(acc, h3, deg, b3, batch3, Wl, bl):
    def body(acc_ref, h_ref, deg_ref, b_ref, batch_ref, wl_ref, bl_ref,
             out_ref, psum, pcnt):
        i = pl.program_id(0)

        @pl.when(i == 0)
        def _():
            psum[...] = jnp.zeros_like(psum)
            pcnt[...] = jnp.zeros_like(pcnt)

        dinv = _dinv_block(deg_ref)
        agg = dinv * (acc_ref[0] + acc_ref[1] + h_ref[...]) + b_ref[...]
        bb = batch_ref[0]
        iota = lax.broadcasted_iota(jnp.int32, (G, RB), 0)
        maskT = (iota == bb).astype(jnp.float32)
        psum[...] += jnp.dot(maskT, agg, preferred_element_type=jnp.float32)
        pcnt[...] += jnp.dot(maskT, jnp.ones((RB, D), jnp.float32),
                             preferred_element_type=jnp.float32)

        @pl.when(i == NBLK - 1)
        def _():
            pooled = psum[...] / jnp.maximum(pcnt[...], 1.0)
            out_ref[...] = jnp.dot(pooled, wl_ref[...],
                                   preferred_element_type=jnp.float32) + bl_ref[...]

    return pl.pallas_call(
        body,
        grid=(NBLK,),
        in_specs=[
            pl.BlockSpec((NCORE, RB, D), lambda i: (0, i, 0)),
            pl.BlockSpec((RB, D), lambda i: (i, 0)),
            pl.BlockSpec((NCORE, RB, DEG_W), lambda i: (0, i, 0)),
            pl.BlockSpec((1, D), lambda i: (0, 0)),
            pl.BlockSpec((1, 1, RB), lambda i: (i, 0, 0)),
            pl.BlockSpec((D, D), lambda i: (0, 0)),
            pl.BlockSpec((1, D), lambda i: (0, 0)),
        ],
        out_specs=pl.BlockSpec((G, D), lambda i: (0, 0)),
        out_shape=jax.ShapeDtypeStruct((G, D), jnp.float32),
        scratch_shapes=[
            pltpu.VMEM((G, D), jnp.float32),
            pltpu.VMEM((G, D), jnp.float32),
        ],
    )(acc, h3, deg, b3, batch3, Wl, bl)


def kernel(x, edge_index, batch, W1, b1, W2, b2, W3, b3, Wlin, blin):
    src = edge_index[0].astype(jnp.int32)
    dst = edge_index[1].astype(jnp.int32)
    e = src.shape[0]
    cpt = _cdiv(e, NW * CH)
    cpt += cpt % 2
    epad = NW * CH * cpt
    src3 = jnp.concatenate(
        [src, jnp.zeros((epad - e,), jnp.int32)]).reshape(NW, cpt, CH)
    dst3 = jnp.concatenate(
        [dst, jnp.full((epad - e,), N, jnp.int32)]).reshape(NW, cpt, CH)

    ones_deg = jnp.ones((CH, DEG_W), jnp.float32)
    zeros_deg = jnp.zeros((ACC_PER_TILE, DEG_W), jnp.float32)
    zeros_acc = jnp.zeros((ACC_PER_TILE, D), jnp.float32)

    deg = _make_deg_kernel(cpt)(dst3, ones_deg, zeros_deg)

    scatter = _make_scatter_kernel(cpt)
    h1 = _tc_lin1(x, W1, deg)
    a1 = scatter(h1, src3, dst3, zeros_acc)
    h2 = _tc_layer(a1, h1, deg, b1.reshape(1, D), W2)
    a2 = scatter(h2, src3, dst3, zeros_acc)
    h3 = _tc_layer(a2, h2, deg, b2.reshape(1, D), W3)
    a3 = scatter(h3, src3, dst3, zeros_acc)

    batch3 = batch.astype(jnp.int32).reshape(NBLK, 1, RB)
    Wl = jnp.zeros((D, D), jnp.float32).at[:, :Wlin.shape[1]].set(Wlin)
    bl = jnp.zeros((1, D), jnp.float32).at[0, :blin.shape[0]].set(blin)
    out = _tc_final(a3, h3, deg, b3.reshape(1, D), batch3, Wl, bl)
    return out[:, :Wlin.shape[1]]


# speedup vs baseline: 1.4827x; 1.4827x over previous
import functools

import jax
import jax.numpy as jnp
from jax import lax
from jax.experimental import pallas as pl
from jax.experimental.pallas import tpu as pltpu
from jax.experimental.pallas import tpu_sc as plsc

N = 10000
D = 128
G = 64
CH = 128
NCORE = 2
NSUB = 16
NW = NCORE * NSUB
ACC_PER_TILE = 640
ACC_ROWS = NSUB * ACC_PER_TILE
DEG_W = 128
RB = 2000
NBLK = N // RB

_sc_mesh = plsc.VectorSubcoreMesh(core_axis_name="c", subcore_axis_name="s")


def _cdiv(a, b):
    return (a + b - 1) // b


@functools.lru_cache(maxsize=None)
def _make_deg_kernel(cpt):
    @functools.partial(
        pl.kernel,
        out_type=jax.ShapeDtypeStruct((NCORE, ACC_ROWS, DEG_W), jnp.float32),
        mesh=_sc_mesh,
        scratch_types=[
            pltpu.VMEM((cpt, CH), jnp.int32),
            pltpu.VMEM((CH, DEG_W), jnp.float32),
            pltpu.VMEM_SHARED((ACC_ROWS, DEG_W), jnp.float32),
        ],
    )
    def deg_kernel(dst_hbm, ones_hbm, zeros_hbm, out_hbm, dst_v, ones_v, acc_sh):
        cid = lax.axis_index("c")
        sid = lax.axis_index("s")
        wid = cid * NSUB + sid
        row0 = sid * ACC_PER_TILE
        pltpu.sync_copy(zeros_hbm, acc_sh.at[pl.ds(row0, ACC_PER_TILE)])
        pltpu.sync_copy(ones_hbm, ones_v)
        pltpu.sync_copy(dst_hbm.at[wid], dst_v)
        plsc.subcore_barrier()

        def body(c, carry):
            pltpu.sync_copy(ones_v, acc_sh.at[dst_v.at[c]], add=True)
            return carry

        lax.fori_loop(0, cpt, body, 0)
        plsc.subcore_barrier()
        pltpu.sync_copy(acc_sh.at[pl.ds(row0, ACC_PER_TILE)],
                        out_hbm.at[cid, pl.ds(row0, ACC_PER_TILE)])

    return deg_kernel


@functools.lru_cache(maxsize=None)
def _make_scatter_kernel(cpt):
    @functools.partial(
        pl.kernel,
        out_type=jax.ShapeDtypeStruct((NCORE, ACC_ROWS, D), jnp.float32),
        mesh=_sc_mesh,
        scratch_types=[
            pltpu.VMEM((cpt, CH), jnp.int32),
            pltpu.VMEM((cpt, CH), jnp.int32),
            pltpu.VMEM((CH, D), jnp.float32),
            pltpu.VMEM_SHARED((ACC_ROWS, D), jnp.float32),
            pltpu.SemaphoreType.DMA,
        ],
    )
    def scatter_kernel(h_hbm, src_hbm, dst_hbm, zeros_hbm, out_hbm,
                       src_v, dst_v, rows_v, acc_sh, sem):
        cid = lax.axis_index("c")
        sid = lax.axis_index("s")
        wid = cid * NSUB + sid
        row0 = sid * ACC_PER_TILE
        pltpu.sync_copy(zeros_hbm, acc_sh.at[pl.ds(row0, ACC_PER_TILE)])
        pltpu.sync_copy(src_hbm.at[wid], src_v)
        pltpu.sync_copy(dst_hbm.at[wid], dst_v)
        plsc.subcore_barrier()

        def body(c, carry):
            pltpu.async_copy(h_hbm.at[src_v.at[c]], rows_v, sem).wait()
            pltpu.sync_copy(rows_v, acc_sh.at[dst_v.at[c]], add=True)
            return carry

        lax.fori_loop(0, cpt, body, 0)
        plsc.subcore_barrier()
        pltpu.sync_copy(acc_sh.at[pl.ds(row0, ACC_PER_TILE)],
                        out_hbm.at[cid, pl.ds(row0, ACC_PER_TILE)])

    return scatter_kernel


def _dinv_block(deg_ref):
    d = deg_ref[0] + deg_ref[1]
    return lax.rsqrt(d[:, 0:1] + 1.0)


def _tc_lin1(x, W1, deg):
    def body(x_ref, w_ref, deg_ref, out_ref):
        dinv = _dinv_block(deg_ref)
        h = jnp.dot(x_ref[...], w_ref[...], preferred_element_type=jnp.float32)
        out_ref[...] = h * dinv

    return pl.pallas_call(
        body,
        grid=(NBLK,),
        in_specs=[
            pl.BlockSpec((RB, D), lambda i: (i, 0)),
            pl.BlockSpec((D, D), lambda i: (0, 0)),
            pl.BlockSpec((NCORE, RB, DEG_W), lambda i: (0, i, 0)),
        ],
        out_specs=pl.BlockSpec((RB, D), lambda i: (i, 0)),
        out_shape=jax.ShapeDtypeStruct((N, D), jnp.float32),
    )(x, W1, deg)


def _tc_layer(acc, hprev, deg, b, W):
    def body(acc_ref, h_ref, deg_ref, b_ref, w_ref, out_ref):
        dinv = _dinv_block(deg_ref)
        agg = dinv * (acc_ref[0] + acc_ref[1] + h_ref[...]) + b_ref[...]
        h = jnp.maximum(agg, 0.0)
        out_ref[...] = dinv * jnp.dot(h, w_ref[...],
                                      preferred_element_type=jnp.float32)

    return pl.pallas_call(
        body,
        grid=(NBLK,),
        in_specs=[
            pl.BlockSpec((NCORE, RB, D), lambda i: (0, i, 0)),
            pl.BlockSpec((RB, D), lambda i: (i, 0)),
            pl.BlockSpec((NCORE, RB, DEG_W), lambda i: (0, i, 0)),
            pl.BlockSpec((1, D), lambda i: (0, 0)),
            pl.BlockSpec((D, D), lambda i: (0, 0)),
        ],
        out_specs=pl.BlockSpec((RB, D), lambda i: (i, 0)),
        out_shape=jax.ShapeDtypeStruct((N, D), jnp.float32),
    )(acc, hprev, deg, b, W)


def _tc_final(acc, h3, deg, b3, batch3, Wl, bl):
    def body(acc_ref, h_ref, deg_ref, b_ref, batch_ref, wl_ref, bl_ref,
             out_ref, psum, pcnt):
        i = pl.program_id(0)

        @pl.when(i == 0)
        def _():
            psum[...] = jnp.zeros_like(psum)
            pcnt[...] = jnp.zeros_like(pcnt)

        dinv = _dinv_block(deg_ref)
        agg = dinv * (acc_ref[0] + acc_ref[1] + h_ref[...]) + b_ref[...]
        bb = batch_ref[0]
        iota = lax.broadcasted_iota(jnp.int32, (G, RB), 0)
        maskT = (iota == bb).astype(jnp.float32)
        psum[...] += jnp.dot(maskT, agg, preferred_element_type=jnp.float32)
        pcnt[...] += jnp.dot(maskT, jnp.ones((RB, D), jnp.float32),
                             preferred_element_type=jnp.float32)

        @pl.when(i == NBLK - 1)
        def _():
            pooled = psum[...] / jnp.maximum(pcnt[...], 1.0)
            out_ref[...] = jnp.dot(pooled, wl_ref[...],
                                   preferred_element_type=jnp.float32) + bl_ref[...]

    return pl.pallas_call(
        body,
        grid=(NBLK,),
        in_specs=[
            pl.BlockSpec((NCORE, RB, D), lambda i: (0, i, 0)),
            pl.BlockSpec((RB, D), lambda i: (i, 0)),
            pl.BlockSpec((NCORE, RB, DEG_W), lambda i: (0, i, 0)),
            pl.BlockSpec((1, D), lambda i: (0, 0)),
            pl.BlockSpec((1, 1, RB), lambda i: (i, 0, 0)),
            pl.BlockSpec((D, D), lambda i: (0, 0)),
            pl.BlockSpec((1, D), lambda i: (0, 0)),
        ],
        out_specs=pl.BlockSpec((G, D), lambda i: (0, 0)),
        out_shape=jax.ShapeDtypeStruct((G, D), jnp.float32),
        scratch_shapes=[
            pltpu.VMEM((G, D), jnp.float32),
            pltpu.VMEM((G, D), jnp.float32),
        ],
    )(acc, h3, deg, b3, batch3, Wl, bl)


def kernel(x, edge_index, batch, W1, b1, W2, b2, W3, b3, Wlin, blin):
    src = edge_index[0].astype(jnp.int32)
    dst = edge_index[1].astype(jnp.int32)
    e = src.shape[0]
    cpt = _cdiv(e, NW * CH)
    epad = NW * CH * cpt
    src3 = jnp.concatenate(
        [src, jnp.zeros((epad - e,), jnp.int32)]).reshape(NW, cpt, CH)
    dst3 = jnp.concatenate(
        [dst, jnp.full((epad - e,), N, jnp.int32)]).reshape(NW, cpt, CH)

    ones_deg = jnp.ones((CH, DEG_W), jnp.float32)
    zeros_deg = jnp.zeros((ACC_PER_TILE, DEG_W), jnp.float32)
    zeros_acc = jnp.zeros((ACC_PER_TILE, D), jnp.float32)

    deg = _make_deg_kernel(cpt)(dst3, ones_deg, zeros_deg)

    scatter = _make_scatter_kernel(cpt)
    h1 = _tc_lin1(x, W1, deg)
    a1 = scatter(h1, src3, dst3, zeros_acc)
    h2 = _tc_layer(a1, h1, deg, b1.reshape(1, D), W2)
    a2 = scatter(h2, src3, dst3, zeros_acc)
    h3 = _tc_layer(a2, h2, deg, b2.reshape(1, D), W3)
    a3 = scatter(h3, src3, dst3, zeros_acc)

    batch3 = batch.astype(jnp.int32).reshape(NBLK, 1, RB)
    Wl = jnp.zeros((D, D), jnp.float32).at[:, :Wlin.shape[1]].set(Wlin)
    bl = jnp.zeros((1, D), jnp.float32).at[0, :blin.shape[0]].set(blin)
    out = _tc_final(a3, h3, deg, b3.reshape(1, D), batch3, Wl, bl)
    return out[:, :Wlin.shape[1]]


# speedup vs baseline: 2.4427x; 1.6475x over previous
import functools

import jax
import jax.numpy as jnp
from jax import lax
from jax.experimental import pallas as pl
from jax.experimental.pallas import tpu as pltpu
from jax.experimental.pallas import tpu_sc as plsc

N = 10000
D = 128
G = 64
CH = 128
NCORE = 2
NSUB = 16
NW = NCORE * NSUB
ACC_PER_TILE = 640
ACC_ROWS = NSUB * ACC_PER_TILE
DEG_W = 128
RB = 2000
NBLK = N // RB

_sc_mesh = plsc.VectorSubcoreMesh(core_axis_name="c", subcore_axis_name="s")


def _cdiv(a, b):
    return (a + b - 1) // b


@functools.lru_cache(maxsize=None)
def _make_deg_kernel(cpt):
    @functools.partial(
        pl.kernel,
        out_type=jax.ShapeDtypeStruct((NCORE, ACC_ROWS, DEG_W), jnp.float32),
        mesh=_sc_mesh,
        scratch_types=[
            pltpu.VMEM((cpt, CH), jnp.int32),
            pltpu.VMEM((CH, DEG_W), jnp.float32),
            pltpu.VMEM_SHARED((ACC_ROWS, DEG_W), jnp.float32),
        ],
    )
    def deg_kernel(dst_hbm, ones_hbm, zeros_hbm, out_hbm, dst_v, ones_v, acc_sh):
        cid = lax.axis_index("c")
        sid = lax.axis_index("s")
        wid = cid * NSUB + sid
        row0 = sid * ACC_PER_TILE
        pltpu.sync_copy(zeros_hbm, acc_sh.at[pl.ds(row0, ACC_PER_TILE)])
        pltpu.sync_copy(ones_hbm, ones_v)
        pltpu.sync_copy(dst_hbm.at[wid], dst_v)
        plsc.subcore_barrier()

        def body(c, carry):
            pltpu.sync_copy(ones_v, acc_sh.at[dst_v.at[c]], add=True)
            return carry

        lax.fori_loop(0, cpt, body, 0)
        plsc.subcore_barrier()
        pltpu.sync_copy(acc_sh.at[pl.ds(row0, ACC_PER_TILE)],
                        out_hbm.at[cid, pl.ds(row0, ACC_PER_TILE)])

    return deg_kernel


@functools.lru_cache(maxsize=None)
def _make_scatter_kernel(cpt):
    @functools.partial(
        pl.kernel,
        out_type=jax.ShapeDtypeStruct((NCORE, ACC_ROWS, D), jnp.float32),
        mesh=_sc_mesh,
        scratch_types=[
            pltpu.VMEM((cpt, CH), jnp.int32),
            pltpu.VMEM((cpt, CH), jnp.int32),
            pltpu.VMEM((CH, D), jnp.float32),
            pltpu.VMEM_SHARED((ACC_ROWS, D), jnp.float32),
            pltpu.SemaphoreType.DMA,
        ],
    )
    def scatter_kernel(h_hbm, src_hbm, dst_hbm, zeros_hbm, out_hbm,
                       src_v, dst_v, rows_v, acc_sh, sem):
        cid = lax.axis_index("c")
        sid = lax.axis_index("s")
        wid = cid * NSUB + sid
        row0 = sid * ACC_PER_TILE
        pltpu.sync_copy(zeros_hbm, acc_sh.at[pl.ds(row0, ACC_PER_TILE)])
        pltpu.sync_copy(src_hbm.at[wid], src_v)
        pltpu.sync_copy(dst_hbm.at[wid], dst_v)
        plsc.subcore_barrier()

        def body(c, carry):
            pltpu.async_copy(h_hbm.at[src_v.at[c]], rows_v, sem).wait()
            pltpu.sync_copy(rows_v, acc_sh.at[dst_v.at[c]], add=True)
            return carry

        lax.fori_loop(0, cpt, body, 0)
        plsc.subcore_barrier()
        pltpu.sync_copy(acc_sh.at[pl.ds(row0, ACC_PER_TILE)],
                        out_hbm.at[cid, pl.ds(row0, ACC_PER_TILE)])

    return scatter_kernel


def _dinv_block(deg_ref):
    d = deg_ref[0] + deg_ref[1]
    return lax.rsqrt(d[:, 0:1] + 1.0)


def _tc_lin1(x, W1, deg):
    def body(x_ref, w_ref, deg_ref, out_ref):
        dinv = _dinv_block(deg_ref)
        h = jnp.dot(x_ref[...], w_ref[...], preferred_element_type=jnp.float32)
        out_ref[...] = h * dinv

    return pl.pallas_call(
        body,
        grid=(NBLK,),
        in_specs=[
            pl.BlockSpec((RB, D), lambda i: (i, 0)),
            pl.BlockSpec((D, D), lambda i: (0, 0)),
            pl.BlockSpec((NCORE, RB, DEG_W), lambda i: (0, i, 0)),
        ],
        out_specs=pl.BlockSpec((RB, D), lambda i: (i, 0)),
        out_shape=jax.ShapeDtypeStruct((N, D), jnp.float32),
    )(x, W1, deg)


def _tc_layer(acc, hprev, deg, b, W):
    def body(acc_ref, h_ref, deg_ref, b_ref, w_ref, out_ref):
        dinv = _dinv_block(deg_ref)
        agg = dinv * (acc_ref[0] + acc_ref[1] + h_ref[...]) + b_ref[...]
        h = jnp.maximum(agg, 0.0)
        out_ref[...] = dinv * jnp.dot(h, w_ref[...],
                                      preferred_element_type=jnp.float32)

    return pl.pallas_call(
        body,
        grid=(NBLK,),
        in_specs=[
            pl.BlockSpec((NCORE, RB, D), lambda i: (0, i, 0)),
            pl.BlockSpec((RB, D), lambda i: (i, 0)),
            pl.BlockSpec((NCORE, RB, DEG_W), lambda i: (0, i, 0)),
            pl.BlockSpec((1, D), lambda i: (0, 0)),
            pl.BlockSpec((D, D), lambda i: (0, 0)),
        ],
        out_specs=pl.BlockSpec((RB, D), lambda i: (i, 0)),
        out_shape=jax.ShapeDtypeStruct((N, D), jnp.float32),
    )(acc, hprev, deg, b, W)


def _tc_final(acc, h3, deg, b3, batch3, Wl, bl):
    def body(acc_ref, h_ref, deg_ref, b_ref, batch_ref, wl_ref, bl_ref,
             out_ref, psum, pcnt):
        i = pl.program_id(0)

        @pl.when(i == 0)
        def _():
            psum[...] = jnp.zeros_like(psum)
            pcnt[...] = jnp.zeros_like(pcnt)

        dinv = _dinv_block(deg_ref)
        agg = dinv * (acc_ref[0] + acc_ref[1] + h_ref[...]) + b_ref[...]
        bb = batch_ref[0]
        iota = lax.broadcasted_iota(jnp.int32, (G, RB), 0)
        maskT = (iota == bb).astype(jnp.float32)
        psum[...] += jnp.dot(maskT, agg, preferred_element_type=jnp.float32)
        pcnt[...] += jnp.dot(maskT, jnp.ones((RB, D), jnp.float32),
                             preferred_element_type=jnp.float32)

        @pl.when(i == NBLK - 1)
        def _():
            pooled = psum[...] / jnp.maximum(pcnt[...], 1.0)
            out_ref[...] = jnp.dot(pooled, wl_ref[...],
                                   preferred_element_type=jnp.float32) + bl_ref[...]

    return pl.pallas_call(
        body,
        grid=(NBLK,),
        in_specs=[
            pl.BlockSpec((NCORE, RB, D), lambda i: (0, i, 0)),
            pl.BlockSpec((RB, D), lambda i: (i, 0)),
            pl.BlockSpec((NCORE, RB, DEG_W), lambda i: (0, i, 0)),
            pl.BlockSpec((1, D), lambda i: (0, 0)),
            pl.BlockSpec((1, 1, RB), lambda i: (i, 0, 0)),
            pl.BlockSpec((D, D), lambda i: (0, 0)),
            pl.BlockSpec((1, D), lambda i: (0, 0)),
        ],
        out_specs=pl.BlockSpec((G, D), lambda i: (0, 0)),
        out_shape=jax.ShapeDtypeStruct((G, D), jnp.float32),
        scratch_shapes=[
            pltpu.VMEM((G, D), jnp.float32),
            pltpu.VMEM((G, D), jnp.float32),
        ],
    )(acc, h3, deg, b3, batch3, Wl, bl)


def kernel(x, edge_index, batch, W1, b1, W2, b2, W3, b3, Wlin, blin):
    src = edge_index[0].astype(jnp.int32)
    dst = edge_index[1].astype(jnp.int32)
    e = src.shape[0]
    cpt = _cdiv(e, NW * CH)
    epad = NW * CH * cpt
    pad_i = jnp.arange(epad - e, dtype=jnp.int32)
    src3 = jnp.concatenate([src, pad_i % N]).reshape(NW, cpt, CH)
    dst3 = jnp.concatenate(
        [dst, N + pad_i % (ACC_ROWS - N)]).reshape(NW, cpt, CH)

    ones_deg = jnp.ones((CH, DEG_W), jnp.float32)
    zeros_deg = jnp.zeros((ACC_PER_TILE, DEG_W), jnp.float32)
    zeros_acc = jnp.zeros((ACC_PER_TILE, D), jnp.float32)

    deg = _make_deg_kernel(cpt)(dst3, ones_deg, zeros_deg)

    scatter = _make_scatter_kernel(cpt)
    h1 = _tc_lin1(x, W1, deg)
    a1 = scatter(h1, src3, dst3, zeros_acc)
    h2 = _tc_layer(a1, h1, deg, b1.reshape(1, D), W2)
    a2 = scatter(h2, src3, dst3, zeros_acc)
    h3 = _tc_layer(a2, h2, deg, b2.reshape(1, D), W3)
    a3 = scatter(h3, src3, dst3, zeros_acc)

    batch3 = batch.astype(jnp.int32).reshape(NBLK, 1, RB)
    Wl = jnp.zeros((D, D), jnp.float32).at[:, :Wlin.shape[1]].set(Wlin)
    bl = jnp.zeros((1, D), jnp.float32).at[0, :blin.shape[0]].set(blin)
    out = _tc_final(a3, h3, deg, b3.reshape(1, D), batch3, Wl, bl)
    return out[:, :Wlin.shape[1]]


# speedup vs baseline: 2.7048x; 1.1073x over previous
import functools

import jax
import jax.numpy as jnp
from jax import lax
from jax.experimental import pallas as pl
from jax.experimental.pallas import tpu as pltpu
from jax.experimental.pallas import tpu_sc as plsc

N = 10000
D = 128
G = 64
CH = 128
NCORE = 2
NSUB = 16
NW = NCORE * NSUB
ACC_PER_TILE = 640
ACC_ROWS = NSUB * ACC_PER_TILE
DEG_W = 128
RB = 2000
NBLK = N // RB

_sc_mesh = plsc.VectorSubcoreMesh(core_axis_name="c", subcore_axis_name="s")


def _cdiv(a, b):
    return (a + b - 1) // b


@functools.lru_cache(maxsize=None)
def _make_deg_kernel(cpt):
    @functools.partial(
        pl.kernel,
        out_type=jax.ShapeDtypeStruct((NCORE, ACC_ROWS, DEG_W), jnp.float32),
        mesh=_sc_mesh,
        scratch_types=[
            pltpu.VMEM((cpt, CH), jnp.int32),
            pltpu.VMEM((CH, DEG_W), jnp.float32),
            pltpu.VMEM_SHARED((ACC_ROWS, DEG_W), jnp.float32),
        ],
    )
    def deg_kernel(dst_hbm, ones_hbm, zeros_hbm, out_hbm, dst_v, ones_v, acc_sh):
        cid = lax.axis_index("c")
        sid = lax.axis_index("s")
        wid = cid * NSUB + sid
        row0 = sid * ACC_PER_TILE
        pltpu.sync_copy(zeros_hbm, acc_sh.at[pl.ds(row0, ACC_PER_TILE)])
        pltpu.sync_copy(ones_hbm, ones_v)
        pltpu.sync_copy(dst_hbm.at[wid], dst_v)
        plsc.subcore_barrier()

        def body(c, carry):
            pltpu.sync_copy(ones_v, acc_sh.at[dst_v.at[c]], add=True)
            return carry

        lax.fori_loop(0, cpt, body, 0)
        plsc.subcore_barrier()
        pltpu.sync_copy(acc_sh.at[pl.ds(row0, ACC_PER_TILE)],
                        out_hbm.at[cid, pl.ds(row0, ACC_PER_TILE)])

    return deg_kernel


@functools.lru_cache(maxsize=None)
def _make_scatter_kernel(cpt):
    @functools.partial(
        pl.kernel,
        out_type=jax.ShapeDtypeStruct((NCORE, ACC_ROWS, D), jnp.float32),
        mesh=_sc_mesh,
        scratch_types=[
            pltpu.VMEM((cpt, CH), jnp.int32),
            pltpu.VMEM((1, CH), jnp.int32),
            pltpu.VMEM((1, CH), jnp.int32),
            pltpu.VMEM((CH, D), jnp.float32),
            pltpu.VMEM((CH, D), jnp.float32),
            pltpu.VMEM_SHARED((ACC_ROWS, D), jnp.float32),
            pltpu.SemaphoreType.DMA,
            pltpu.SemaphoreType.DMA,
            pltpu.SemaphoreType.DMA,
            pltpu.SemaphoreType.DMA,
        ],
    )
    def scatter_kernel(h_hbm, src_hbm, dst_hbm, zeros_hbm, out_hbm,
                       src_v, didx_a, didx_b, rows_a, rows_b,
                       acc_sh, gsem_a, gsem_b, isem_a, isem_b):
        cid = lax.axis_index("c")
        sid = lax.axis_index("s")
        wid = cid * NSUB + sid
        row0 = sid * ACC_PER_TILE
        pltpu.sync_copy(zeros_hbm, acc_sh.at[pl.ds(row0, ACC_PER_TILE)])
        pltpu.sync_copy(src_hbm.at[wid], src_v)
        plsc.subcore_barrier()

        def pair(p, carry):
            c = 2 * p
            pltpu.async_copy(dst_hbm.at[wid, pl.ds(c, 1)], didx_a, isem_a)
            pltpu.async_copy(dst_hbm.at[wid, pl.ds(c + 1, 1)], didx_b, isem_b)
            pltpu.async_copy(h_hbm.at[src_v.at[c]], rows_a, gsem_a)
            pltpu.async_copy(h_hbm.at[src_v.at[c + 1]], rows_b, gsem_b)
            pltpu.make_async_copy(h_hbm.at[src_v.at[c]], rows_a, gsem_a).wait()
            pltpu.make_async_copy(h_hbm.at[src_v.at[c + 1]], rows_b, gsem_b).wait()
            pltpu.make_async_copy(dst_hbm.at[0, pl.ds(0, 1)], didx_a, isem_a).wait()
            pltpu.sync_copy(rows_a, acc_sh.at[didx_a.at[0]], add=True)
            pltpu.make_async_copy(dst_hbm.at[0, pl.ds(0, 1)], didx_b, isem_b).wait()
            pltpu.sync_copy(rows_b, acc_sh.at[didx_b.at[0]], add=True)
            return carry

        lax.fori_loop(0, cpt // 2, pair, 0)
        plsc.subcore_barrier()
        pltpu.sync_copy(acc_sh.at[pl.ds(row0, ACC_PER_TILE)],
                        out_hbm.at[cid, pl.ds(row0, ACC_PER_TILE)])

    return scatter_kernel


def _dinv_block(deg_ref):
    d = deg_ref[0] + deg_ref[1]
    return lax.rsqrt(d[:, 0:1] + 1.0)


def _tc_lin1(x, W1, deg):
    def body(x_ref, w_ref, deg_ref, out_ref):
        dinv = _dinv_block(deg_ref)
        h = jnp.dot(x_ref[...], w_ref[...], preferred_element_type=jnp.float32)
        out_ref[...] = h * dinv

    return pl.pallas_call(
        body,
        grid=(NBLK,),
        in_specs=[
            pl.BlockSpec((RB, D), lambda i: (i, 0)),
            pl.BlockSpec((D, D), lambda i: (0, 0)),
            pl.BlockSpec((NCORE, RB, DEG_W), lambda i: (0, i, 0)),
        ],
        out_specs=pl.BlockSpec((RB, D), lambda i: (i, 0)),
        out_shape=jax.ShapeDtypeStruct((N, D), jnp.float32),
    )(x, W1, deg)


def _tc_layer(acc, hprev, deg, b, W):
    def body(acc_ref, h_ref, deg_ref, b_ref, w_ref, out_ref):
        dinv = _dinv_block(deg_ref)
        agg = dinv * (acc_ref[0] + acc_ref[1] + h_ref[...]) + b_ref[...]
        h = jnp.maximum(agg, 0.0)
        out_ref[...] = dinv * jnp.dot(h, w_ref[...],
                                      preferred_element_type=jnp.float32)

    return pl.pallas_call(
        body,
        grid=(NBLK,),
        in_specs=[
            pl.BlockSpec((NCORE, RB, D), lambda i: (0, i, 0)),
            pl.BlockSpec((RB, D), lambda i: (i, 0)),
            pl.BlockSpec((NCORE, RB, DEG_W), lambda i: (0, i, 0)),
            pl.BlockSpec((1, D), lambda i: (0, 0)),
            pl.BlockSpec((D, D), lambda i: (0, 0)),
        ],
        out_specs=pl.BlockSpec((RB, D), lambda i: (i, 0)),
        out_shape=jax.ShapeDtypeStruct((N, D), jnp.float32),
    )(acc, hprev, deg, b, W)


def _tc_final(acc, h3, deg, b3, batch3, Wl, bl):
    def body(acc_ref, h_ref, deg_ref, b_ref, batch_ref, wl_ref, bl_ref,
             out_ref, psum, pcnt):
        i = pl.program_id(0)

        @pl.when(i == 0)
        def _():
            psum[...] = jnp.zeros_like(psum)
            pcnt[...] = jnp.zeros_like(pcnt)

        dinv = _dinv_block(deg_ref)
        agg = dinv * (acc_ref[0] + acc_ref[1] + h_ref[...]) + b_ref[...]
        bb = batch_ref[0]
        iota = lax.broadcasted_iota(jnp.int32, (G, RB), 0)
        maskT = (iota == bb).astype(jnp.float32)
        psum[...] += jnp.dot(maskT, agg, preferred_element_type=jnp.float32)
        pcnt[...] += jnp.dot(maskT, jnp.ones((RB, D), jnp.float32),
                             preferred_element_type=jnp.float32)

        @pl.when(i == NBLK - 1)
        def _():
            pooled = psum[...] / jnp.maximum(pcnt[...], 1.0)
            out_ref[...] = jnp.dot(pooled, wl_ref[...],
                                   preferred_element_type=jnp.float32) + bl_ref[...]

    return pl.pallas_call(
        body,
        grid=(NBLK,),
        in_specs=[
            pl.BlockSpec((NCORE, RB, D), lambda i: (0, i, 0)),
            pl.BlockSpec((RB, D), lambda i: (i, 0)),
            pl.BlockSpec((NCORE, RB, DEG_W), lambda i: (0, i, 0)),
            pl.BlockSpec((1, D), lambda i: (0, 0)),
            pl.BlockSpec((1, 1, RB), lambda i: (i, 0, 0)),
            pl.BlockSpec((D, D), lambda i: (0, 0)),
            pl.BlockSpec((1, D), lambda i: (0, 0)),
        ],
        out_specs=pl.BlockSpec((G, D), lambda i: (0, 0)),
        out_shape=jax.ShapeDtypeStruct((G, D), jnp.float32),
        scratch_shapes=[
            pltpu.VMEM((G, D), jnp.float32),
            pltpu.VMEM((G, D), jnp.float32),
        ],
    )(acc, h3, deg, b3, batch3, Wl, bl)


def kernel(x, edge_index, batch, W1, b1, W2, b2, W3, b3, Wlin, blin):
    src = edge_index[0].astype(jnp.int32)
    dst = edge_index[1].astype(jnp.int32)
    e = src.shape[0]
    cpt = _cdiv(e, NW * CH)
    cpt += cpt % 2
    epad = NW * CH * cpt
    pad_i = jnp.arange(epad - e, dtype=jnp.int32)
    src3 = jnp.concatenate([src, pad_i % N]).reshape(NW, cpt, CH)
    dst3 = jnp.concatenate(
        [dst, N + pad_i % (ACC_ROWS - N)]).reshape(NW, cpt, CH)

    ones_deg = jnp.ones((CH, DEG_W), jnp.float32)
    zeros_deg = jnp.zeros((ACC_PER_TILE, DEG_W), jnp.float32)
    zeros_acc = jnp.zeros((ACC_PER_TILE, D), jnp.float32)

    deg = _make_deg_kernel(cpt)(dst3, ones_deg, zeros_deg)

    scatter = _make_scatter_kernel(cpt)
    h1 = _tc_lin1(x, W1, deg)
    a1 = scatter(h1, src3, dst3, zeros_acc)
    h2 = _tc_layer(a1, h1, deg, b1.reshape(1, D), W2)
    a2 = scatter(h2, src3, dst3, zeros_acc)
    h3 = _tc_layer(a2, h2, deg, b2.reshape(1, D), W3)
    a3 = scatter(h3, src3, dst3, zeros_acc)

    batch3 = batch.astype(jnp.int32).reshape(NBLK, 1, RB)
    Wl = jnp.zeros((D, D), jnp.float32).at[:, :Wlin.shape[1]].set(Wlin)
    bl = jnp.zeros((1, D), jnp.float32).at[0, :blin.shape[0]].set(blin)
    out = _tc_final(a3, h3, deg, b3.reshape(1, D), batch3, Wl, bl)
    return out[:, :Wlin.shape[1]]


# speedup vs baseline: 2.7362x; 1.0116x over previous
import functools

import jax
import jax.numpy as jnp
from jax import lax
from jax.experimental import pallas as pl
from jax.experimental.pallas import tpu as pltpu
from jax.experimental.pallas import tpu_sc as plsc

N = 10000
D = 128
G = 64
CH = 128
NCORE = 2
NSUB = 16
NW = NCORE * NSUB
ACC_PER_TILE = 640
ACC_ROWS = NSUB * ACC_PER_TILE
DEG_W = 128
RB = 2000
NBLK = N // RB

_sc_mesh = plsc.VectorSubcoreMesh(core_axis_name="c", subcore_axis_name="s")


def _cdiv(a, b):
    return (a + b - 1) // b


@functools.lru_cache(maxsize=None)
def _make_deg_kernel(cpt):
    @functools.partial(
        pl.kernel,
        out_type=jax.ShapeDtypeStruct((NCORE, ACC_ROWS, DEG_W), jnp.float32),
        mesh=_sc_mesh,
        scratch_types=[
            pltpu.VMEM((cpt, CH), jnp.int32),
            pltpu.VMEM((CH, DEG_W), jnp.float32),
            pltpu.VMEM_SHARED((ACC_ROWS, DEG_W), jnp.float32),
            pltpu.SemaphoreType.DMA,
            pltpu.SemaphoreType.DMA,
        ],
    )
    def deg_kernel(dst_hbm, ones_hbm, zeros_hbm, out_hbm, dst_v, ones_v, acc_sh,
                   ssem_a, ssem_b):
        cid = lax.axis_index("c")
        sid = lax.axis_index("s")
        wid = cid * NSUB + sid
        row0 = sid * ACC_PER_TILE
        pltpu.sync_copy(zeros_hbm, acc_sh.at[pl.ds(row0, ACC_PER_TILE)])
        pltpu.sync_copy(ones_hbm, ones_v)
        pltpu.sync_copy(dst_hbm.at[wid], dst_v)
        plsc.subcore_barrier()

        def pair(p, carry):
            c = 2 * p
            pltpu.async_copy(ones_v, acc_sh.at[dst_v.at[c]], ssem_a, add=True)
            pltpu.async_copy(ones_v, acc_sh.at[dst_v.at[c + 1]], ssem_b, add=True)
            pltpu.make_async_copy(ones_v, acc_sh.at[dst_v.at[c]], ssem_a).wait()
            pltpu.make_async_copy(ones_v, acc_sh.at[dst_v.at[c + 1]], ssem_b).wait()
            return carry

        lax.fori_loop(0, cpt // 2, pair, 0)
        plsc.subcore_barrier()
        pltpu.sync_copy(acc_sh.at[pl.ds(row0, ACC_PER_TILE)],
                        out_hbm.at[cid, pl.ds(row0, ACC_PER_TILE)])

    return deg_kernel


@functools.lru_cache(maxsize=None)
def _make_scatter_kernel(cpt):
    @functools.partial(
        pl.kernel,
        out_type=jax.ShapeDtypeStruct((NCORE, ACC_ROWS, D), jnp.float32),
        mesh=_sc_mesh,
        scratch_types=[
            pltpu.VMEM((cpt, CH), jnp.int32),
            pltpu.VMEM((1, CH), jnp.int32),
            pltpu.VMEM((1, CH), jnp.int32),
            pltpu.VMEM((CH, D), jnp.float32),
            pltpu.VMEM((CH, D), jnp.float32),
            pltpu.VMEM_SHARED((ACC_ROWS, D), jnp.float32),
            pltpu.SemaphoreType.DMA,
            pltpu.SemaphoreType.DMA,
            pltpu.SemaphoreType.DMA,
            pltpu.SemaphoreType.DMA,
            pltpu.SemaphoreType.DMA,
            pltpu.SemaphoreType.DMA,
        ],
    )
    def scatter_kernel(h_hbm, src_hbm, dst_hbm, zeros_hbm, out_hbm,
                       src_v, didx_a, didx_b, rows_a, rows_b,
                       acc_sh, gsem_a, gsem_b, isem_a, isem_b, ssem_a, ssem_b):
        cid = lax.axis_index("c")
        sid = lax.axis_index("s")
        wid = cid * NSUB + sid
        row0 = sid * ACC_PER_TILE
        pltpu.sync_copy(zeros_hbm, acc_sh.at[pl.ds(row0, ACC_PER_TILE)])
        pltpu.sync_copy(src_hbm.at[wid], src_v)
        plsc.subcore_barrier()

        def pair(p, carry):
            c = 2 * p
            pltpu.async_copy(dst_hbm.at[wid, pl.ds(c, 1)], didx_a, isem_a)
            pltpu.async_copy(dst_hbm.at[wid, pl.ds(c + 1, 1)], didx_b, isem_b)
            pltpu.async_copy(h_hbm.at[src_v.at[c]], rows_a, gsem_a)
            pltpu.async_copy(h_hbm.at[src_v.at[c + 1]], rows_b, gsem_b)
            pltpu.make_async_copy(h_hbm.at[src_v.at[c]], rows_a, gsem_a).wait()
            pltpu.make_async_copy(h_hbm.at[src_v.at[c + 1]], rows_b, gsem_b).wait()
            pltpu.make_async_copy(dst_hbm.at[0, pl.ds(0, 1)], didx_a, isem_a).wait()
            pltpu.make_async_copy(dst_hbm.at[0, pl.ds(0, 1)], didx_b, isem_b).wait()
            pltpu.async_copy(rows_a, acc_sh.at[didx_a.at[0]], ssem_a, add=True)
            pltpu.async_copy(rows_b, acc_sh.at[didx_b.at[0]], ssem_b, add=True)
            pltpu.make_async_copy(rows_a, acc_sh.at[didx_a.at[0]], ssem_a).wait()
            pltpu.make_async_copy(rows_b, acc_sh.at[didx_b.at[0]], ssem_b).wait()
            return carry

        lax.fori_loop(0, cpt // 2, pair, 0)
        plsc.subcore_barrier()
        pltpu.sync_copy(acc_sh.at[pl.ds(row0, ACC_PER_TILE)],
                        out_hbm.at[cid, pl.ds(row0, ACC_PER_TILE)])

    return scatter_kernel


def _dinv_block(deg_ref):
    d = deg_ref[0] + deg_ref[1]
    return lax.rsqrt(d[:, 0:1] + 1.0)


def _tc_lin1(x, W1, deg):
    def body(x_ref, w_ref, deg_ref, out_ref):
        dinv = _dinv_block(deg_ref)
        h = jnp.dot(x_ref[...], w_ref[...], preferred_element_type=jnp.float32)
        out_ref[...] = h * dinv

    return pl.pallas_call(
        body,
        grid=(NBLK,),
        in_specs=[
            pl.BlockSpec((RB, D), lambda i: (i, 0)),
            pl.BlockSpec((D, D), lambda i: (0, 0)),
            pl.BlockSpec((NCORE, RB, DEG_W), lambda i: (0, i, 0)),
        ],
        out_specs=pl.BlockSpec((RB, D), lambda i: (i, 0)),
        out_shape=jax.ShapeDtypeStruct((N, D), jnp.float32),
    )(x, W1, deg)


def _tc_layer(acc, hprev, deg, b, W):
    def body(acc_ref, h_ref, deg_ref, b_ref, w_ref, out_ref):
        dinv = _dinv_block(deg_ref)
        agg = dinv * (acc_ref[0] + acc_ref[1] + h_ref[...]) + b_ref[...]
        h = jnp.maximum(agg, 0.0)
        out_ref[...] = dinv * jnp.dot(h, w_ref[...],
                                      preferred_element_type=jnp.float32)

    return pl.pallas_call(
        body,
        grid=(NBLK,),
        in_specs=[
            pl.BlockSpec((NCORE, RB, D), lambda i: (0, i, 0)),
            pl.BlockSpec((RB, D), lambda i: (i, 0)),
            pl.BlockSpec((NCORE, RB, DEG_W), lambda i: (0, i, 0)),
            pl.BlockSpec((1, D), lambda i: (0, 0)),
            pl.BlockSpec((D, D), lambda i: (0, 0)),
        ],
        out_specs=pl.BlockSpec((RB, D), lambda i: (i, 0)),
        out_shape=jax.ShapeDtypeStruct((N, D), jnp.float32),
    )(acc, hprev, deg, b, W)


def _tc_final(acc, h3, deg, b3, batch3, Wl, bl):
    def body(acc_ref, h_ref, deg_ref, b_ref, batch_ref, wl_ref, bl_ref,
             out_ref, psum, pcnt):
        i = pl.program_id(0)

        @pl.when(i == 0)
        def _():
            psum[...] = jnp.zeros_like(psum)
            pcnt[...] = jnp.zeros_like(pcnt)

        dinv = _dinv_block(deg_ref)
        agg = dinv * (acc_ref[0] + acc_ref[1] + h_ref[...]) + b_ref[...]
        bb = batch_ref[0]
        iota = lax.broadcasted_iota(jnp.int32, (G, RB), 0)
        maskT = (iota == bb).astype(jnp.float32)
        psum[...] += jnp.dot(maskT, agg, preferred_element_type=jnp.float32)
        pcnt[...] += jnp.dot(maskT, jnp.ones((RB, D), jnp.float32),
                             preferred_element_type=jnp.float32)

        @pl.when(i == NBLK - 1)
        def _():
            pooled = psum[...] / jnp.maximum(pcnt[...], 1.0)
            out_ref[...] = jnp.dot(pooled, wl_ref[...],
                                   preferred_element_type=jnp.float32) + bl_ref[...]

    return pl.pallas_call(
        body,
        grid=(NBLK,),
        in_specs=[
            pl.BlockSpec((NCORE, RB, D), lambda i: (0, i, 0)),
            pl.BlockSpec((RB, D), lambda i: (i, 0)),
            pl.BlockSpec((NCORE, RB, DEG_W), lambda i: (0, i, 0)),
            pl.BlockSpec((1, D), lambda i: (0, 0)),
            pl.BlockSpec((1, 1, RB), lambda i: (i, 0, 0)),
            pl.BlockSpec((D, D), lambda i: (0, 0)),
            pl.BlockSpec((1, D), lambda i: (0, 0)),
        ],
        out_specs=pl.BlockSpec((G, D), lambda i: (0, 0)),
        out_shape=jax.ShapeDtypeStruct((G, D), jnp.float32),
        scratch_shapes=[
            pltpu.VMEM((G, D), jnp.float32),
            pltpu.VMEM((G, D), jnp.float32),
        ],
    )(acc, h3, deg, b3, batch3, Wl, bl)


def kernel(x, edge_index, batch, W1, b1, W2, b2, W3, b3, Wlin, blin):
    src = edge_index[0].astype(jnp.int32)
    dst = edge_index[1].astype(jnp.int32)
    e = src.shape[0]
    cpt = _cdiv(e, NW * CH)
    cpt += cpt % 2
    epad = NW * CH * cpt
    pad_i = jnp.arange(epad - e, dtype=jnp.int32)
    src3 = jnp.concatenate([src, pad_i % N]).reshape(NW, cpt, CH)
    dst3 = jnp.concatenate(
        [dst, N + pad_i % (ACC_ROWS - N)]).reshape(NW, cpt, CH)

    ones_deg = jnp.ones((CH, DEG_W), jnp.float32)
    zeros_deg = jnp.zeros((ACC_PER_TILE, DEG_W), jnp.float32)
    zeros_acc = jnp.zeros((ACC_PER_TILE, D), jnp.float32)

    deg = _make_deg_kernel(cpt)(dst3, ones_deg, zeros_deg)

    scatter = _make_scatter_kernel(cpt)
    h1 = _tc_lin1(x, W1, deg)
    a1 = scatter(h1, src3, dst3, zeros_acc)
    h2 = _tc_layer(a1, h1, deg, b1.reshape(1, D), W2)
    a2 = scatter(h2, src3, dst3, zeros_acc)
    h3 = _tc_layer(a2, h2, deg, b2.reshape(1, D), W3)
    a3 = scatter(h3, src3, dst3, zeros_acc)

    batch3 = batch.astype(jnp.int32).reshape(NBLK, 1, RB)
    Wl = jnp.zeros((D, D), jnp.float32).at[:, :Wlin.shape[1]].set(Wlin)
    bl = jnp.zeros((1, D), jnp.float32).at[0, :blin.shape[0]].set(blin)
    out = _tc_final(a3, h3, deg, b3.reshape(1, D), batch3, Wl, bl)
    return out[:, :Wlin.shape[1]]
